# baseline (speedup 1.0000x reference)
import functools

import numpy as np
import jax
import jax.numpy as jnp
from jax.experimental import pallas as pl
from jax.experimental.pallas import tpu as pltpu


def _build_scatter(H, W):
    H2, W2 = 2 * H, 2 * W
    E = np.zeros((4, H * W, H2 * W2), np.float32)
    for i in range(H2):
        for j in range(W2):
            p = (i % 2) * 2 + (j % 2)
            E[p, (i // 2) * W + (j // 2), i * W2 + j] = 1.0
    return jnp.asarray(E.reshape(4 * H * W, H2 * W2), jnp.bfloat16)


def _unet_fused_kernel(u_ref, x_ref, wstack_ref, bup_ref, e_ref,
                       w1_ref, b1_ref, w2_ref, b2_ref,
                       out_ref, sum_ref, ssq_ref, *, W2, Co, Cout, NB):
    HW2 = x_ref.shape[-1]
    lane = jax.lax.broadcasted_iota(jnp.int32, (1, HW2), 1)
    not_row0 = lane >= W2
    not_rowL = lane < (HW2 - W2)
    not_col0 = (lane % W2) != 0
    not_colL = (lane % W2) != (W2 - 1)

    def conv3x3_relu(xb, w_all, b_col, C):
        pk = pltpu.bitcast(xb, jnp.float32)
        v_up = jnp.where(not_row0, pltpu.roll(pk, shift=W2, axis=1), 0.0)
        v_dn = jnp.where(not_rowL, pltpu.roll(pk, shift=HW2 - W2, axis=1), 0.0)
        rhs3 = jnp.concatenate([pltpu.bitcast(v_up, jnp.bfloat16), xb,
                                pltpu.bitcast(v_dn, jnp.bfloat16)], axis=0)
        y = jnp.dot(w_all, rhs3,
                    preferred_element_type=jnp.float32)
        acc = y[1 * C:2 * C] + b_col
        left = pltpu.roll(y[0 * C:1 * C], shift=1, axis=1)
        acc = acc + jnp.where(not_col0, left, 0.0)
        right = pltpu.roll(y[2 * C:3 * C], shift=HW2 - 1, axis=1)
        acc = acc + jnp.where(not_colL, right, 0.0)
        return jnp.maximum(acc, 0.0)

    for i in range(NB):
        ub = u_ref[i].astype(jnp.bfloat16)
        z = jnp.dot(wstack_ref[...], ub,
                    preferred_element_type=jnp.float32)
        zb = z.astype(jnp.bfloat16)
        zcat = jnp.concatenate([zb[p * Co:(p + 1) * Co] for p in range(4)],
                               axis=1)
        up = jnp.dot(zcat, e_ref[...],
                     preferred_element_type=jnp.float32)
        up = up + bup_ref[...]

        cat_b = jnp.concatenate([up.astype(jnp.bfloat16),
                                 x_ref[i].astype(jnp.bfloat16)], axis=0)

        h1 = conv3x3_relu(cat_b, w1_ref[...], b1_ref[...], Cout)
        h2 = conv3x3_relu(h1.astype(jnp.bfloat16), w2_ref[...], b2_ref[...],
                          Cout)

        out_ref[i] = h2.astype(jnp.bfloat16)
        sum_ref[i] = jnp.sum(h2, axis=1, keepdims=True)
        ssq_ref[i] = jnp.sum(h2 * h2, axis=1, keepdims=True)


def _bn_apply_kernel(h_ref, scale_ref, shift_ref, out_ref):
    out_ref[...] = (h_ref[...].astype(jnp.float32) * scale_ref[...]
                    + shift_ref[...])


@jax.jit
def _unet_block_forward(up_in, x_in, params):
    B, Cup, H, W = up_in.shape
    Co = Cup // 2
    _, Cx, H2, W2 = x_in.shape
    assert H2 == 2 * H and W2 == 2 * W
    Cmid = Cx + Co
    Cout = Cmid // 2
    HW = H * W
    HW2 = H2 * W2
    eps = 1e-5
    assert Cup % 8 == 0 and Co % 8 == 0 and Cx % 8 == 0 and Cout % 8 == 0

    u_flat = up_in.reshape(B, Cup, HW)
    x_flat = x_in.reshape(B, Cx, HW2)

    wstack = jnp.transpose(params["w_up"], (2, 3, 1, 0)) \
        .reshape(4 * Co, Cup).astype(jnp.bfloat16)
    bup = params["b_up"].reshape(Co, 1)

    w1_all = jnp.transpose(params["w1"], (3, 0, 2, 1)) \
        .reshape(3 * Cout, 3 * Cmid).astype(jnp.bfloat16)
    b1 = params["b1"].reshape(Cout, 1)
    w2_all = jnp.transpose(params["w2"], (3, 0, 2, 1)) \
        .reshape(3 * Cout, 3 * Cout).astype(jnp.bfloat16)
    b2 = params["b2"].reshape(Cout, 1)
    gamma = params["gamma"].reshape(Cout, 1)
    beta = params["beta"].reshape(Cout, 1)

    emat = _build_scatter(H, W)

    cparams = pltpu.CompilerParams(
        dimension_semantics=("parallel",),
        vmem_limit_bytes=48 * 1024 * 1024,
    )

    NB1 = 4 if B % 8 == 0 else 1
    kernel1 = functools.partial(_unet_fused_kernel, W2=W2, Co=Co, Cout=Cout,
                                NB=NB1)
    h2, psum, psq = pl.pallas_call(
        kernel1,
        out_shape=(jax.ShapeDtypeStruct((B, Cout, HW2), jnp.bfloat16),
                   jax.ShapeDtypeStruct((B, Cout, 1), jnp.float32),
                   jax.ShapeDtypeStruct((B, Cout, 1), jnp.float32)),
        grid=(B // NB1,),
        in_specs=[
            pl.BlockSpec((NB1, Cup, HW), lambda b: (b, 0, 0)),
            pl.BlockSpec((NB1, Cx, HW2), lambda b: (b, 0, 0)),
            pl.BlockSpec((4 * Co, Cup), lambda b: (0, 0)),
            pl.BlockSpec((Co, 1), lambda b: (0, 0)),
            pl.BlockSpec((4 * HW, HW2), lambda b: (0, 0)),
            pl.BlockSpec((3 * Cout, 3 * Cmid), lambda b: (0, 0)),
            pl.BlockSpec((Cout, 1), lambda b: (0, 0)),
            pl.BlockSpec((3 * Cout, 3 * Cout), lambda b: (0, 0)),
            pl.BlockSpec((Cout, 1), lambda b: (0, 0)),
        ],
        out_specs=(pl.BlockSpec((NB1, Cout, HW2), lambda b: (b, 0, 0)),
                   pl.BlockSpec((NB1, Cout, 1), lambda b: (b, 0, 0)),
                   pl.BlockSpec((NB1, Cout, 1), lambda b: (b, 0, 0))),
        compiler_params=cparams,
    )(u_flat, x_flat, wstack, bup, emat, w1_all, b1, w2_all, b2)

    count = B * HW2
    mean = jnp.sum(psum, axis=0) / count
    var = jnp.sum(psq, axis=0) / count - mean * mean
    scale = gamma * jax.lax.rsqrt(var + eps)
    shift = beta - mean * scale

    NB = 4 if B % 4 == 0 else 1
    out = pl.pallas_call(
        _bn_apply_kernel,
        out_shape=jax.ShapeDtypeStruct((B, Cout, HW2), jnp.float32),
        grid=(B // NB,),
        in_specs=[pl.BlockSpec((NB, Cout, HW2), lambda b: (b, 0, 0)),
                  pl.BlockSpec((Cout, 1), lambda b: (0, 0)),
                  pl.BlockSpec((Cout, 1), lambda b: (0, 0))],
        out_specs=pl.BlockSpec((NB, Cout, HW2), lambda b: (b, 0, 0)),
        compiler_params=cparams,
    )(h2, scale, shift)

    return out.reshape(B, Cout, H2, W2)


def kernel(up_in, x_in, w_up, b_up, w1, b1, w2, b2, gamma, beta):
    params = {"w_up": w_up, "b_up": b_up, "w1": w1, "b1": b1,
              "w2": w2, "b2": b2, "gamma": gamma, "beta": beta}
    return _unet_block_forward(up_in, x_in, params)


# speedup vs baseline: 1.0364x; 1.0364x over previous
import functools

import numpy as np
import jax
import jax.numpy as jnp
from jax.experimental import pallas as pl
from jax.experimental.pallas import tpu as pltpu


def _build_scatter(H, W):
    H2, W2 = 2 * H, 2 * W
    E = np.zeros((4, H * W, H2 * W2), np.float32)
    for i in range(H2):
        for j in range(W2):
            p = (i % 2) * 2 + (j % 2)
            E[p, (i // 2) * W + (j // 2), i * W2 + j] = 1.0
    return jnp.asarray(E.reshape(4 * H * W, H2 * W2), jnp.bfloat16)


def _unet_fused_kernel(u_ref, x_ref, wstack_ref, bup_ref, e_ref,
                       w1_ref, b1_ref, w2_ref, b2_ref,
                       out_ref, sum_ref, ssq_ref, *, W2, Co, Cout, NB):
    HW2 = x_ref.shape[-1]
    lane = jax.lax.broadcasted_iota(jnp.int32, (1, HW2), 1)
    tap_keep = []
    for t in range(9):
        dh, dw = t // 3, t % 3
        keep = None
        if dh == 0:
            keep = lane >= W2
        elif dh == 2:
            keep = lane < (HW2 - W2)
        if dw == 0:
            cm = (lane % W2) != 0
            keep = cm if keep is None else jnp.logical_and(keep, cm)
        elif dw == 2:
            cm = (lane % W2) != (W2 - 1)
            keep = cm if keep is None else jnp.logical_and(keep, cm)
        tap_keep.append(keep)

    def conv3x3_relu(xb, w_all, b_col, C):
        y = jnp.dot(w_all, xb,
                    preferred_element_type=jnp.float32)
        yb = y.astype(jnp.bfloat16)
        acc = y[4 * C:5 * C] + b_col
        for t in range(9):
            if t == 4:
                continue
            dh, dw = t // 3, t % 3
            off = (dh - 1) * W2 + (dw - 1)
            packed = pltpu.bitcast(yb[t * C:(t + 1) * C], jnp.float32)
            rolled = pltpu.roll(packed, shift=(-off) % HW2, axis=1)
            kept = jnp.where(tap_keep[t], rolled, 0.0)
            acc = acc + pltpu.bitcast(kept,
                                      jnp.bfloat16).astype(jnp.float32)
        return jnp.maximum(acc, 0.0)

    for i in range(NB):
        ub = u_ref[i].astype(jnp.bfloat16)
        z = jnp.dot(wstack_ref[...], ub,
                    preferred_element_type=jnp.float32)
        zb = z.astype(jnp.bfloat16)
        zcat = jnp.concatenate([zb[p * Co:(p + 1) * Co] for p in range(4)],
                               axis=1)
        up = jnp.dot(zcat, e_ref[...],
                     preferred_element_type=jnp.float32)
        up = up + bup_ref[...]

        cat_b = jnp.concatenate([up.astype(jnp.bfloat16),
                                 x_ref[i].astype(jnp.bfloat16)], axis=0)

        h1 = conv3x3_relu(cat_b, w1_ref[...], b1_ref[...], Cout)
        h2 = conv3x3_relu(h1.astype(jnp.bfloat16), w2_ref[...], b2_ref[...],
                          Cout)

        out_ref[i] = h2.astype(jnp.bfloat16)
        sum_ref[i] = jnp.sum(h2, axis=1, keepdims=True)
        ssq_ref[i] = jnp.sum(h2 * h2, axis=1, keepdims=True)


def _bn_apply_kernel(h_ref, scale_ref, shift_ref, out_ref):
    out_ref[...] = (h_ref[...].astype(jnp.float32) * scale_ref[...]
                    + shift_ref[...])


@jax.jit
def _unet_block_forward(up_in, x_in, params):
    B, Cup, H, W = up_in.shape
    Co = Cup // 2
    _, Cx, H2, W2 = x_in.shape
    assert H2 == 2 * H and W2 == 2 * W
    Cmid = Cx + Co
    Cout = Cmid // 2
    HW = H * W
    HW2 = H2 * W2
    eps = 1e-5
    assert Cup % 8 == 0 and Co % 8 == 0 and Cx % 8 == 0 and Cout % 8 == 0

    u_flat = up_in.reshape(B, Cup, HW)
    x_flat = x_in.reshape(B, Cx, HW2)

    wstack = jnp.transpose(params["w_up"], (2, 3, 1, 0)) \
        .reshape(4 * Co, Cup).astype(jnp.bfloat16)
    bup = params["b_up"].reshape(Co, 1)

    w1_all = jnp.transpose(params["w1"], (2, 3, 0, 1)) \
        .reshape(9 * Cout, Cmid).astype(jnp.bfloat16)
    b1 = params["b1"].reshape(Cout, 1)
    w2_all = jnp.transpose(params["w2"], (2, 3, 0, 1)) \
        .reshape(9 * Cout, Cout).astype(jnp.bfloat16)
    b2 = params["b2"].reshape(Cout, 1)
    gamma = params["gamma"].reshape(Cout, 1)
    beta = params["beta"].reshape(Cout, 1)

    emat = _build_scatter(H, W)

    cparams = pltpu.CompilerParams(
        dimension_semantics=("parallel",),
        vmem_limit_bytes=48 * 1024 * 1024,
    )

    NB1 = 4 if B % 8 == 0 else 1
    kernel1 = functools.partial(_unet_fused_kernel, W2=W2, Co=Co, Cout=Cout,
                                NB=NB1)
    h2, psum, psq = pl.pallas_call(
        kernel1,
        out_shape=(jax.ShapeDtypeStruct((B, Cout, HW2), jnp.bfloat16),
                   jax.ShapeDtypeStruct((B, Cout, 1), jnp.float32),
                   jax.ShapeDtypeStruct((B, Cout, 1), jnp.float32)),
        grid=(B // NB1,),
        in_specs=[
            pl.BlockSpec((NB1, Cup, HW), lambda b: (b, 0, 0)),
            pl.BlockSpec((NB1, Cx, HW2), lambda b: (b, 0, 0)),
            pl.BlockSpec((4 * Co, Cup), lambda b: (0, 0)),
            pl.BlockSpec((Co, 1), lambda b: (0, 0)),
            pl.BlockSpec((4 * HW, HW2), lambda b: (0, 0)),
            pl.BlockSpec((9 * Cout, Cmid), lambda b: (0, 0)),
            pl.BlockSpec((Cout, 1), lambda b: (0, 0)),
            pl.BlockSpec((9 * Cout, Cout), lambda b: (0, 0)),
            pl.BlockSpec((Cout, 1), lambda b: (0, 0)),
        ],
        out_specs=(pl.BlockSpec((NB1, Cout, HW2), lambda b: (b, 0, 0)),
                   pl.BlockSpec((NB1, Cout, 1), lambda b: (b, 0, 0)),
                   pl.BlockSpec((NB1, Cout, 1), lambda b: (b, 0, 0))),
        compiler_params=cparams,
    )(u_flat, x_flat, wstack, bup, emat, w1_all, b1, w2_all, b2)

    count = B * HW2
    mean = jnp.sum(psum, axis=0) / count
    var = jnp.sum(psq, axis=0) / count - mean * mean
    scale = gamma * jax.lax.rsqrt(var + eps)
    shift = beta - mean * scale

    NB = 4 if B % 4 == 0 else 1
    out = pl.pallas_call(
        _bn_apply_kernel,
        out_shape=jax.ShapeDtypeStruct((B, Cout, HW2), jnp.float32),
        grid=(B // NB,),
        in_specs=[pl.BlockSpec((NB, Cout, HW2), lambda b: (b, 0, 0)),
                  pl.BlockSpec((Cout, 1), lambda b: (0, 0)),
                  pl.BlockSpec((Cout, 1), lambda b: (0, 0))],
        out_specs=pl.BlockSpec((NB, Cout, HW2), lambda b: (b, 0, 0)),
        compiler_params=cparams,
    )(h2, scale, shift)

    return out.reshape(B, Cout, H2, W2)


def kernel(up_in, x_in, w_up, b_up, w1, b1, w2, b2, gamma, beta):
    params = {"w_up": w_up, "b_up": b_up, "w1": w1, "b1": b1,
              "w2": w2, "b2": b2, "gamma": gamma, "beta": beta}
    return _unet_block_forward(up_in, x_in, params)
